# baseline (speedup 1.0000x reference)
import functools

import jax
import jax.numpy as jnp
from jax.experimental import pallas as pl
from jax.experimental.pallas import tpu as pltpu

_TOP_K = 2
_AUX_W = 0.01


def _fused_body(x_ref, w_ref, aux_ref, idx_ref, score_ref, acc_ref, *, inv_hw):
    j = pl.program_id(0)

    @pl.when(j == 0)
    def _init():
        acc_ref[...] = jnp.zeros_like(acc_ref)

    xb = x_ref[...]
    hwb = xb.shape[1]
    part = xb[:, 0:8, :]
    for k in range(1, hwb // 8):
        part = part + xb[:, 8 * k:8 * (k + 1), :]
    acc_ref[...] += part

    @pl.when(j == pl.num_programs(0) - 1)
    def _finish():
        b8, _, dim = acc_ref.shape
        acc2 = acc_ref[...].reshape(b8 * 8, dim)
        y = jax.lax.dot_general(
            acc2, w_ref[...], (((1,), (1,)), ((), ())),
            preferred_element_type=jnp.float32)
        logits = jnp.sum(y.reshape(b8, 8, y.shape[-1]), axis=1) * inv_hw
        b, e = logits.shape
        m = jnp.max(logits, axis=-1, keepdims=True)
        ex = jnp.exp(logits - m)
        sm = ex / jnp.sum(ex, axis=-1, keepdims=True)

        col = jax.lax.broadcasted_iota(jnp.int32, (b, e), 1)
        s1 = jnp.max(sm, axis=-1, keepdims=True)
        idx1 = jnp.min(jnp.where(sm == s1, col, e), axis=-1, keepdims=True)
        masked = jnp.where(col == idx1, -jnp.inf, sm)
        s2 = jnp.max(masked, axis=-1, keepdims=True)
        idx2 = jnp.min(jnp.where(masked == s2, col, e), axis=-1, keepdims=True)

        onehot = ((col == idx1) | (col == idx2)).astype(jnp.float32)
        importance = jnp.mean(sm, axis=0)
        load = jnp.mean(onehot, axis=0) / _TOP_K
        aux_ref[...] = jnp.full(
            (1, 1), _AUX_W * float(e * e), jnp.float32) * jnp.mean(
                importance * load)

        k_col = jax.lax.broadcasted_iota(jnp.int32, (b, _TOP_K), 1)
        idx_ref[...] = jnp.where(k_col == 0, idx1, idx2).astype(jnp.int32)
        score_ref[...] = jnp.where(k_col == 0, s1, s2)


def kernel(x, W_gate):
    b, h, w, dim = x.shape
    e = W_gate.shape[0]
    hw = h * w
    x3 = x.reshape(b, hw, dim)

    hwb = 48
    grid = (hw // hwb,)

    aux, idx, scores = pl.pallas_call(
        functools.partial(_fused_body, inv_hw=1.0 / hw),
        grid=grid,
        in_specs=[
            pl.BlockSpec((b, hwb, dim), lambda j: (0, j, 0)),
            pl.BlockSpec((e, dim), lambda j: (0, 0)),
        ],
        out_specs=(
            pl.BlockSpec((1, 1), lambda j: (0, 0)),
            pl.BlockSpec((b, _TOP_K), lambda j: (0, 0)),
            pl.BlockSpec((b, _TOP_K), lambda j: (0, 0)),
        ),
        out_shape=(
            jax.ShapeDtypeStruct((1, 1), jnp.float32),
            jax.ShapeDtypeStruct((b, _TOP_K), jnp.int32),
            jax.ShapeDtypeStruct((b, _TOP_K), jnp.float32),
        ),
        scratch_shapes=[pltpu.VMEM((b, 8, dim), jnp.float32)],
        compiler_params=pltpu.CompilerParams(
            dimension_semantics=("arbitrary",)),
    )(x3, W_gate)

    return aux.reshape(()), idx, scores


# speedup vs baseline: 1.0085x; 1.0085x over previous
import functools

import jax
import jax.numpy as jnp
from jax.experimental import pallas as pl
from jax.experimental.pallas import tpu as pltpu

_TOP_K = 2
_AUX_W = 0.01


def _fused_body(x_ref, w_ref, aux_ref, idx_ref, score_ref, logit_acc, *,
                inv_hw):
    i = pl.program_id(0)
    xb = x_ref[...]
    bb, hw, dim = xb.shape
    part = jnp.sum(xb.reshape(bb, hw // 8, 8, dim), axis=1)
    y = jax.lax.dot_general(
        part.reshape(bb * 8, dim), w_ref[...], (((1,), (1,)), ((), ())),
        preferred_element_type=jnp.float32)
    rows = jnp.sum(y.reshape(bb, 8, y.shape[-1]), axis=1) * inv_hw
    logit_acc[pl.ds(i * bb, bb), :] = rows

    @pl.when(i == pl.num_programs(0) - 1)
    def _finish():
        logits = logit_acc[...]
        b, e = logits.shape
        m = jnp.max(logits, axis=-1, keepdims=True)
        ex = jnp.exp(logits - m)
        sm = ex / jnp.sum(ex, axis=-1, keepdims=True)

        col = jax.lax.broadcasted_iota(jnp.int32, (b, e), 1)
        s1 = jnp.max(sm, axis=-1, keepdims=True)
        idx1 = jnp.min(jnp.where(sm == s1, col, e), axis=-1, keepdims=True)
        masked = jnp.where(col == idx1, -jnp.inf, sm)
        s2 = jnp.max(masked, axis=-1, keepdims=True)
        idx2 = jnp.min(jnp.where(masked == s2, col, e), axis=-1, keepdims=True)

        onehot = ((col == idx1) | (col == idx2)).astype(jnp.float32)
        importance = jnp.mean(sm, axis=0)
        load = jnp.mean(onehot, axis=0) / _TOP_K
        aux_ref[...] = jnp.full(
            (1, 1), _AUX_W * float(e * e), jnp.float32) * jnp.mean(
                importance * load)

        k_col = jax.lax.broadcasted_iota(jnp.int32, (b, _TOP_K), 1)
        idx_ref[...] = jnp.where(k_col == 0, idx1, idx2).astype(jnp.int32)
        score_ref[...] = jnp.where(k_col == 0, s1, s2)


def kernel(x, W_gate):
    b, h, w, dim = x.shape
    e = W_gate.shape[0]
    hw = h * w
    x3 = x.reshape(b, hw, dim)

    bb = 8
    grid = (b // bb,)

    aux, idx, scores = pl.pallas_call(
        functools.partial(_fused_body, inv_hw=1.0 / hw),
        grid=grid,
        in_specs=[
            pl.BlockSpec((bb, hw, dim), lambda i: (i, 0, 0)),
            pl.BlockSpec((e, dim), lambda i: (0, 0)),
        ],
        out_specs=(
            pl.BlockSpec((1, 1), lambda i: (0, 0)),
            pl.BlockSpec((b, _TOP_K), lambda i: (0, 0)),
            pl.BlockSpec((b, _TOP_K), lambda i: (0, 0)),
        ),
        out_shape=(
            jax.ShapeDtypeStruct((1, 1), jnp.float32),
            jax.ShapeDtypeStruct((b, _TOP_K), jnp.int32),
            jax.ShapeDtypeStruct((b, _TOP_K), jnp.float32),
        ),
        scratch_shapes=[pltpu.VMEM((b, e), jnp.float32)],
        compiler_params=pltpu.CompilerParams(
            dimension_semantics=("arbitrary",)),
    )(x3, W_gate)

    return aux.reshape(()), idx, scores


# speedup vs baseline: 1.0202x; 1.0116x over previous
import functools

import jax
import jax.numpy as jnp
from jax.experimental import pallas as pl
from jax.experimental.pallas import tpu as pltpu

_TOP_K = 2
_AUX_W = 0.01


def _fused_body(x_ref, w_ref, aux_ref, idx_ref, score_ref, logit_acc, *,
                inv_hw):
    i = pl.program_id(0)
    xb = x_ref[...]
    bb, hw, dim = xb.shape
    part = jnp.sum(xb.reshape(bb, hw // 8, 8, dim), axis=1)
    y = jax.lax.dot_general(
        part.reshape(bb * 8, dim), w_ref[...], (((1,), (1,)), ((), ())),
        preferred_element_type=jnp.float32)
    rows = jnp.sum(y.reshape(bb, 8, y.shape[-1]), axis=1) * inv_hw
    logit_acc[pl.ds(i * bb, bb), :] = rows

    @pl.when(i == pl.num_programs(0) - 1)
    def _finish():
        logits = logit_acc[...]
        b, e = logits.shape
        m = jnp.max(logits, axis=-1, keepdims=True)
        ex = jnp.exp(logits - m)
        sm = ex / jnp.sum(ex, axis=-1, keepdims=True)

        col = jax.lax.broadcasted_iota(jnp.int32, (b, e), 1)
        s1 = jnp.max(sm, axis=-1, keepdims=True)
        idx1 = jnp.min(jnp.where(sm == s1, col, e), axis=-1, keepdims=True)
        masked = jnp.where(col == idx1, -jnp.inf, sm)
        s2 = jnp.max(masked, axis=-1, keepdims=True)
        idx2 = jnp.min(jnp.where(masked == s2, col, e), axis=-1, keepdims=True)

        onehot = ((col == idx1) | (col == idx2)).astype(jnp.float32)
        importance = jnp.mean(sm, axis=0)
        load = jnp.mean(onehot, axis=0) / _TOP_K
        aux_ref[...] = jnp.full(
            (1, 1), _AUX_W * float(e * e), jnp.float32) * jnp.mean(
                importance * load)

        k_col = jax.lax.broadcasted_iota(jnp.int32, (b, _TOP_K), 1)
        idx_ref[...] = jnp.where(k_col == 0, idx1, idx2).astype(jnp.int32)
        score_ref[...] = jnp.where(k_col == 0, s1, s2)


def kernel(x, W_gate):
    b, h, w, dim = x.shape
    e = W_gate.shape[0]
    hw = h * w
    x3 = x.reshape(b, hw, dim)

    bb = 4
    grid = (b // bb,)

    aux, idx, scores = pl.pallas_call(
        functools.partial(_fused_body, inv_hw=1.0 / hw),
        grid=grid,
        in_specs=[
            pl.BlockSpec((bb, hw, dim), lambda i: (i, 0, 0)),
            pl.BlockSpec((e, dim), lambda i: (0, 0)),
        ],
        out_specs=(
            pl.BlockSpec((1, 1), lambda i: (0, 0)),
            pl.BlockSpec((b, _TOP_K), lambda i: (0, 0)),
            pl.BlockSpec((b, _TOP_K), lambda i: (0, 0)),
        ),
        out_shape=(
            jax.ShapeDtypeStruct((1, 1), jnp.float32),
            jax.ShapeDtypeStruct((b, _TOP_K), jnp.int32),
            jax.ShapeDtypeStruct((b, _TOP_K), jnp.float32),
        ),
        scratch_shapes=[pltpu.VMEM((b, e), jnp.float32)],
        compiler_params=pltpu.CompilerParams(
            dimension_semantics=("arbitrary",)),
    )(x3, W_gate)

    return aux.reshape(()), idx, scores
